# Initial kernel scaffold; baseline (speedup 1.0000x reference)
import jax
import jax.numpy as jnp
from jax.experimental import pallas as pl


def kernel(x, fifo_buffer, norm_w, mem_W1, mem_W2, slots, usage, in_proj_w, in_proj_b, attn_out_w, attn_out_b, fus_w, fus_b, outp_w):
    raise NotImplementedError("write your pallas kernel here")


# speedup vs baseline: 2.7390x; 2.7390x over previous
import jax
import jax.numpy as jnp
from jax.experimental import pallas as pl
from jax.experimental.pallas import tpu as pltpu

B, L, D, H, S, FIFO = 4, 2048, 1024, 256, 32, 512
NHEADS = 4
DH = D // NHEADS
TEMP = 0.35
EPS = 1e-6
BL = 512
NL = L // BL

_F32 = jnp.float32


def _dott(a, b):
    return jax.lax.dot_general(a, b, (((1,), (1,)), ((), ())),
                               preferred_element_type=_F32)


def _fused_kernel(x_ref, fifo_ref, norm_w_ref, w1_ref, w2_ref, slots_ref,
                  qkv_w_ref, qkv_b_ref, ao_w_ref, ao_b_ref, fus_w_ref,
                  fus_b_ref, outp_w_ref, out_ref, nf_ref, k_scr, v_scr):
    l = pl.program_id(1)

    @pl.when(l == 0)
    def _():
        fifo = fifo_ref[0]
        k_scr[...] = _dott(fifo, qkv_w_ref[D:2 * D, :]) + qkv_b_ref[0, D:2 * D][None, :]
        v_scr[...] = _dott(fifo, qkv_w_ref[2 * D:3 * D, :]) + qkv_b_ref[0, 2 * D:3 * D][None, :]

    xb = x_ref[0]
    x_norm = xb / jnp.sqrt(jnp.mean(xb * xb, axis=-1, keepdims=True) + EPS)
    x_norm = x_norm * norm_w_ref[0][None, :]

    hpre = _dott(x_norm, w1_ref[...])
    hmid = 0.5 * hpre * (1.0 + jax.lax.erf(hpre * (2.0 ** -0.5)))
    lt = _dott(hmid, w2_ref[...])

    slots = slots_ref[...]
    slots_n = slots / jnp.maximum(
        jnp.sqrt(jnp.sum(slots * slots, axis=-1, keepdims=True)), 1e-12)
    qn = x_norm / jnp.maximum(
        jnp.sqrt(jnp.sum(x_norm * x_norm, axis=-1, keepdims=True)), 1e-12)
    sw = jax.nn.softmax(_dott(qn, slots_n) / TEMP, axis=-1)
    summary = jnp.dot(sw, slots, preferred_element_type=_F32)

    q = _dott(x_norm, qkv_w_ref[0:D, :]) + qkv_b_ref[0, 0:D][None, :]
    scale = 1.0 / jnp.sqrt(jnp.float32(DH))
    heads = []
    for hh in range(NHEADS):
        qh = q[:, hh * DH:(hh + 1) * DH]
        kh = k_scr[:, hh * DH:(hh + 1) * DH]
        vh = v_scr[:, hh * DH:(hh + 1) * DH]
        att = jax.nn.softmax(_dott(qh, kh) * scale, axis=-1)
        heads.append(jnp.dot(att, vh, preferred_element_type=_F32))
    st = jnp.concatenate(heads, axis=-1)
    st = _dott(st, ao_w_ref[...]) + ao_b_ref[0][None, :]

    fw = fus_w_ref[...]
    logits = (_dott(x_norm, fw[:, 0:D]) + _dott(st, fw[:, D:2 * D])
              + _dott(lt, fw[:, 2 * D:3 * D]) + _dott(summary, fw[:, 3 * D:4 * D])
              + fus_b_ref[0][None, :])
    g = jax.nn.softmax(logits, axis=-1)
    fused = g[:, 0:1] * st + g[:, 1:2] * lt + g[:, 2:3] * summary

    out_ref[0] = _dott(fused, outp_w_ref[...]) + xb
    nf_ref[0] = x_norm


def kernel(x, fifo_buffer, norm_w, mem_W1, mem_W2, slots, usage, in_proj_w,
           in_proj_b, attn_out_w, attn_out_b, fus_w, fus_b, outp_w):
    del usage
    const = lambda b, l: (0, 0)
    out, new_fifo = pl.pallas_call(
        _fused_kernel,
        grid=(B, NL),
        in_specs=[
            pl.BlockSpec((1, BL, D), lambda b, l: (b, l, 0)),
            pl.BlockSpec((1, FIFO, D), lambda b, l: (b, 0, 0)),
            pl.BlockSpec((1, D), const),
            pl.BlockSpec((H, D), const),
            pl.BlockSpec((D, H), const),
            pl.BlockSpec((S, D), const),
            pl.BlockSpec((3 * D, D), const),
            pl.BlockSpec((1, 3 * D), const),
            pl.BlockSpec((D, D), const),
            pl.BlockSpec((1, D), const),
            pl.BlockSpec((3, 4 * D), const),
            pl.BlockSpec((1, 3), const),
            pl.BlockSpec((D, D), const),
        ],
        out_specs=[
            pl.BlockSpec((1, BL, D), lambda b, l: (b, l, 0)),
            pl.BlockSpec((1, FIFO, D), lambda b, l: (b, 0, 0)),
        ],
        out_shape=[
            jax.ShapeDtypeStruct((B, L, D), _F32),
            jax.ShapeDtypeStruct((B, FIFO, D), _F32),
        ],
        scratch_shapes=[
            pltpu.VMEM((FIFO, D), _F32),
            pltpu.VMEM((FIFO, D), _F32),
        ],
        compiler_params=pltpu.CompilerParams(
            dimension_semantics=("arbitrary", "arbitrary"),
            vmem_limit_bytes=128 * 1024 * 1024,
        ),
    )(x, fifo_buffer, norm_w.reshape(1, D), mem_W1, mem_W2, slots,
      in_proj_w, in_proj_b.reshape(1, 3 * D), attn_out_w,
      attn_out_b.reshape(1, D), fus_w, fus_b.reshape(1, 3), outp_w)
    return out, new_fifo
